# baseline (device time: 28704 ns/iter reference)
import os

import jax
import jax.numpy as jnp
from jax import lax
from jax.experimental import pallas as pl
from jax.experimental.pallas import tpu as pltpu

N_DEV = 16
C_GLOBAL = 8192
EPS = 1e-5

if os.environ.get("COLLECTIVE", "2round") == "a2a":
    ROUNDS = (tuple(range(1, 16)),)
else:
    ROUNDS = ((1, 2, 3), (4, 8, 12))
N_ROUNDS = len(ROUNDS)
MAXP = max(len(offs) for offs in ROUNDS)
_NCOMM = int(os.environ.get("BISECT_STEPS", str(N_ROUNDS)))
N_CHUNKS = int(os.environ.get("NCHUNKS", "4"))


def kernel(x, t_emb, W_scale, W_shift):
    B, S, C_loc = x.shape
    T = B * S
    CH = T // N_CHUNKS

    def body(x_hbm, t_ref, ws_ref, wsh_ref, out_ref,
             xf_ref, acc_ref, recv_ref, stage_ref,
             load_sems, send_sems, recv_sems, store_sems):
        my = lax.axis_index("i")

        barrier_sem = pltpu.get_barrier_semaphore()
        n_partners = 0
        for offs in ROUNDS:
            for off in offs:
                pl.semaphore_signal(
                    barrier_sem, inc=1,
                    device_id=(jnp.bitwise_xor(my, off),),
                    device_id_type=pl.DeviceIdType.MESH,
                )
                n_partners += 1

        def rowslice(c):
            return divmod(c * CH, S)

        def chunk_load(c):
            b, r0 = rowslice(c)
            return pltpu.make_async_copy(
                x_hbm.at[b, pl.ds(r0, CH), :],
                xf_ref.at[b, pl.ds(r0, CH), :],
                load_sems.at[c % 2],
            )

        def make_rdma(c, r, j, off):
            return pltpu.make_async_remote_copy(
                src_ref=acc_ref.at[c, r],
                dst_ref=recv_ref.at[c, r, j],
                send_sem=send_sems.at[c, r, j],
                recv_sem=recv_sems.at[c, r, j],
                device_id=(jnp.bitwise_xor(my, off),),
                device_id_type=pl.DeviceIdType.MESH,
            )

        loads = {0: chunk_load(0)}
        loads[0].start()
        rdmas = {}
        cur = {}
        stores = {}

        def fire_round(c, r):
            if r >= _NCOMM:
                return
            for j, off in enumerate(ROUNDS[r]):
                rdmas[(c, r, j)] = make_rdma(c, r, j, off)
                rdmas[(c, r, j)].start()

        def stage_a(c):
            if c + 1 < N_CHUNKS:
                loads[c + 1] = chunk_load(c + 1)
                loads[c + 1].start()
            loads[c].wait()
            b, r0 = rowslice(c)
            v = xf_ref[b, r0:r0 + CH, :]
            stats_t = jnp.concatenate(
                [jnp.sum(v, axis=1, keepdims=True),
                 jnp.sum(v * v, axis=1, keepdims=True)], axis=1).T
            cur[c] = stats_t
            acc_ref[c, 0] = stats_t.astype(jnp.bfloat16)
            if c == 0:
                pl.semaphore_wait(barrier_sem, n_partners)
            fire_round(c, 0)

        def reduce_round(c, r):
            if r >= _NCOMM:
                return
            for j in range(len(ROUNDS[r])):
                rdmas[(c, r, j)].wait_recv()
            acc = cur[c]
            for j in range(len(ROUNDS[r])):
                acc = acc + recv_ref[c, r, j].astype(jnp.float32)
            cur[c] = acc

        def stage_b(c):
            reduce_round(c, 0)
            acc_ref[c, 1] = cur[c].astype(jnp.bfloat16)
            fire_round(c, 1)

        def stage_c(c, scale, shift):
            reduce_round(c, N_ROUNDS - 1)
            cols = cur[c].T
            mean = cols[:, 0:1] / C_GLOBAL
            ex2 = cols[:, 1:2] / C_GLOBAL
            inv = lax.rsqrt(ex2 - mean * mean + EPS)
            b, r0 = rowslice(c)
            mb = mean.astype(jnp.bfloat16)
            ib = inv.astype(jnp.bfloat16)
            g = (1.0 + scale[b:b + 1, :]).astype(jnp.bfloat16)
            sh = shift[b:b + 1, :].astype(jnp.bfloat16)
            if c >= 2:
                stores[c - 2].wait()
            xb = xf_ref[b, r0:r0 + CH, :].astype(jnp.bfloat16)
            stage_ref[c % 2] = (xb - mb) * ib * g + sh
            stores[c] = pltpu.make_async_copy(
                stage_ref.at[c % 2],
                out_ref.at[b, pl.ds(r0, CH), :],
                store_sems.at[c % 2],
            )
            stores[c].start()

        stage_a(0)
        scale = jnp.dot(t_ref[:], ws_ref[:], preferred_element_type=jnp.float32)
        shift = jnp.dot(t_ref[:], wsh_ref[:], preferred_element_type=jnp.float32)
        if N_ROUNDS == 1:
            for c in range(1, N_CHUNKS):
                stage_a(c)
            for c in range(N_CHUNKS):
                stage_c(c, scale, shift)
        else:
            if N_CHUNKS > 1:
                stage_a(1)
            for c in range(2, N_CHUNKS):
                stage_b(c - 2)
                stage_a(c)
            if N_CHUNKS > 1:
                stage_b(N_CHUNKS - 2)
                stage_c(0, scale, shift)
                stage_b(N_CHUNKS - 1)
                stage_c(1, scale, shift)
            else:
                stage_b(0)
                stage_c(0, scale, shift)
            for c in range(2, N_CHUNKS):
                stage_c(c, scale, shift)

        for key, rdma in rdmas.items():
            rdma.wait_send()
        for c in range(max(N_CHUNKS - 2, 0), N_CHUNKS):
            stores[c].wait()

    return pl.pallas_call(
        body,
        out_shape=jax.ShapeDtypeStruct((B, S, C_loc), jnp.bfloat16),
        in_specs=[
            pl.BlockSpec(memory_space=pl.ANY),
            pl.BlockSpec(memory_space=pltpu.VMEM),
            pl.BlockSpec(memory_space=pltpu.VMEM),
            pl.BlockSpec(memory_space=pltpu.VMEM),
        ],
        out_specs=pl.BlockSpec(memory_space=pl.ANY),
        scratch_shapes=[
            pltpu.VMEM((B, S, C_loc), jnp.float32),
            pltpu.VMEM((N_CHUNKS, N_ROUNDS, 2, T // N_CHUNKS), jnp.bfloat16),
            pltpu.VMEM((N_CHUNKS, N_ROUNDS, MAXP, 2, T // N_CHUNKS),
                       jnp.bfloat16),
            pltpu.VMEM((2, T // N_CHUNKS, C_loc), jnp.bfloat16),
            pltpu.SemaphoreType.DMA((2,)),
            pltpu.SemaphoreType.DMA((N_CHUNKS, N_ROUNDS, MAXP)),
            pltpu.SemaphoreType.DMA((N_CHUNKS, N_ROUNDS, MAXP)),
            pltpu.SemaphoreType.DMA((2,)),
        ],
        compiler_params=pltpu.CompilerParams(
            collective_id=0, vmem_limit_bytes=64 * 1024 * 1024),
    )(x, t_emb, W_scale, W_shift)


# device time: 28093 ns/iter; 1.0217x vs baseline; 1.0217x over previous
import os

import jax
import jax.numpy as jnp
from jax import lax
from jax.experimental import pallas as pl
from jax.experimental.pallas import tpu as pltpu

N_DEV = 16
C_GLOBAL = 8192
EPS = 1e-5

if os.environ.get("COLLECTIVE", "2round") == "a2a":
    ROUNDS = (tuple(range(1, 16)),)
else:
    ROUNDS = ((1, 2, 3), (4, 8, 12))
N_ROUNDS = len(ROUNDS)
MAXP = max(len(offs) for offs in ROUNDS)
_NCOMM = int(os.environ.get("BISECT_STEPS", str(N_ROUNDS)))
N_CHUNKS = int(os.environ.get("NCHUNKS", "4"))


def kernel(x, t_emb, W_scale, W_shift):
    B, S, C_loc = x.shape
    T = B * S
    CH = T // N_CHUNKS

    def body(x_hbm, t_ref, ws_ref, wsh_ref, out_ref,
             xf_ref, xb_ref, acc_ref, recv_ref, stage_ref,
             load_sems, send_sems, recv_sems, store_sems):
        my = lax.axis_index("i")

        barrier_sem = pltpu.get_barrier_semaphore()
        n_partners = 0
        for offs in ROUNDS:
            for off in offs:
                pl.semaphore_signal(
                    barrier_sem, inc=1,
                    device_id=(jnp.bitwise_xor(my, off),),
                    device_id_type=pl.DeviceIdType.MESH,
                )
                n_partners += 1

        def rowslice(c):
            return divmod(c * CH, S)

        def chunk_load(c):
            b, r0 = rowslice(c)
            return pltpu.make_async_copy(
                x_hbm.at[b, pl.ds(r0, CH), :],
                xf_ref.at[b, pl.ds(r0, CH), :],
                load_sems.at[c],
            )

        def make_rdma(c, r, j, off):
            return pltpu.make_async_remote_copy(
                src_ref=acc_ref.at[c, r],
                dst_ref=recv_ref.at[c, r, j],
                send_sem=send_sems.at[c, r, j],
                recv_sem=recv_sems.at[c, r, j],
                device_id=(jnp.bitwise_xor(my, off),),
                device_id_type=pl.DeviceIdType.MESH,
            )

        loads = {}
        for c in range(N_CHUNKS):
            loads[c] = chunk_load(c)
            loads[c].start()
        rdmas = {}
        cur = {}
        stores = {}

        def fire_round(c, r):
            if r >= _NCOMM:
                return
            for j, off in enumerate(ROUNDS[r]):
                rdmas[(c, r, j)] = make_rdma(c, r, j, off)
                rdmas[(c, r, j)].start()

        def stage_a(c):
            loads[c].wait()
            b, r0 = rowslice(c)
            v = xf_ref[b, r0:r0 + CH, :]
            stats_t = jnp.concatenate(
                [jnp.sum(v, axis=1, keepdims=True),
                 jnp.sum(v * v, axis=1, keepdims=True)], axis=1).T
            cur[c] = stats_t
            acc_ref[c, 0] = stats_t.astype(jnp.bfloat16)
            if c == 0:
                pl.semaphore_wait(barrier_sem, n_partners)
            fire_round(c, 0)
            xb_ref[b, r0:r0 + CH, :] = v.astype(jnp.bfloat16)

        def reduce_round(c, r):
            if r >= _NCOMM:
                return
            for j in range(len(ROUNDS[r])):
                rdmas[(c, r, j)].wait_recv()
            acc = cur[c]
            for j in range(len(ROUNDS[r])):
                acc = acc + recv_ref[c, r, j].astype(jnp.float32)
            cur[c] = acc

        def stage_b(c):
            reduce_round(c, 0)
            acc_ref[c, 1] = cur[c].astype(jnp.bfloat16)
            fire_round(c, 1)

        def stage_c(c, scale, shift):
            reduce_round(c, N_ROUNDS - 1)
            cols = cur[c].T
            mean = cols[:, 0:1] / C_GLOBAL
            ex2 = cols[:, 1:2] / C_GLOBAL
            inv = lax.rsqrt(ex2 - mean * mean + EPS)
            b, r0 = rowslice(c)
            mb = mean.astype(jnp.bfloat16)
            ib = inv.astype(jnp.bfloat16)
            g = (1.0 + scale[b:b + 1, :]).astype(jnp.bfloat16)
            sh = shift[b:b + 1, :].astype(jnp.bfloat16)
            if c >= 2:
                stores[c - 2].wait()
            xb = xb_ref[b, r0:r0 + CH, :]
            stage_ref[c % 2] = (xb - mb) * ib * g + sh
            stores[c] = pltpu.make_async_copy(
                stage_ref.at[c % 2],
                out_ref.at[b, pl.ds(r0, CH), :],
                store_sems.at[c % 2],
            )
            stores[c].start()

        stage_a(0)
        scale = jnp.dot(t_ref[:], ws_ref[:], preferred_element_type=jnp.float32)
        shift = jnp.dot(t_ref[:], wsh_ref[:], preferred_element_type=jnp.float32)
        if N_ROUNDS == 1:
            for c in range(1, N_CHUNKS):
                stage_a(c)
            for c in range(N_CHUNKS):
                stage_c(c, scale, shift)
        else:
            if N_CHUNKS > 1:
                stage_a(1)
            for c in range(2, N_CHUNKS):
                stage_b(c - 2)
                stage_a(c)
            if N_CHUNKS > 1:
                stage_b(N_CHUNKS - 2)
                stage_c(0, scale, shift)
                stage_b(N_CHUNKS - 1)
                stage_c(1, scale, shift)
            else:
                stage_b(0)
                stage_c(0, scale, shift)
            for c in range(2, N_CHUNKS):
                stage_c(c, scale, shift)

        for key, rdma in rdmas.items():
            rdma.wait_send()
        for c in range(max(N_CHUNKS - 2, 0), N_CHUNKS):
            stores[c].wait()

    return pl.pallas_call(
        body,
        out_shape=jax.ShapeDtypeStruct((B, S, C_loc), jnp.bfloat16),
        in_specs=[
            pl.BlockSpec(memory_space=pl.ANY),
            pl.BlockSpec(memory_space=pltpu.VMEM),
            pl.BlockSpec(memory_space=pltpu.VMEM),
            pl.BlockSpec(memory_space=pltpu.VMEM),
        ],
        out_specs=pl.BlockSpec(memory_space=pl.ANY),
        scratch_shapes=[
            pltpu.VMEM((B, S, C_loc), jnp.float32),
            pltpu.VMEM((B, S, C_loc), jnp.bfloat16),
            pltpu.VMEM((N_CHUNKS, N_ROUNDS, 2, T // N_CHUNKS), jnp.bfloat16),
            pltpu.VMEM((N_CHUNKS, N_ROUNDS, MAXP, 2, T // N_CHUNKS),
                       jnp.bfloat16),
            pltpu.VMEM((2, T // N_CHUNKS, C_loc), jnp.bfloat16),
            pltpu.SemaphoreType.DMA((N_CHUNKS,)),
            pltpu.SemaphoreType.DMA((N_CHUNKS, N_ROUNDS, MAXP)),
            pltpu.SemaphoreType.DMA((N_CHUNKS, N_ROUNDS, MAXP)),
            pltpu.SemaphoreType.DMA((2,)),
        ],
        compiler_params=pltpu.CompilerParams(
            collective_id=0, vmem_limit_bytes=64 * 1024 * 1024),
    )(x, t_emb, W_scale, W_shift)
